# baseline (speedup 1.0000x reference)
import jax
import jax.numpy as jnp
from jax import lax
from jax.experimental import pallas as pl
from jax.experimental.pallas import tpu as pltpu
from jax.experimental.pallas import tpu_sc as plsc

NUM_CORES = 2
NUM_SUBCORES = 16
LANES = 16
NW = NUM_CORES * NUM_SUBCORES

BATCH = 16384
EMBED_DIM = 32
ROW_W = 128
B_PER_W = BATCH // NW
HALF = B_PER_W // 2


def _mf_kernel(user_ids, item_ids, user_rows, item_rows, user_bias,
               item_bias, global_bias, out_hbm,
               uidx_v, iidx_v, urows_v, irows_v, ub_v, ib_v, gb_v, out_v,
               sem):
    wid = lax.axis_index("s") * NUM_CORES + lax.axis_index("c")
    base = wid * B_PER_W

    pltpu.sync_copy(user_ids.at[pl.ds(base, B_PER_W)], uidx_v)
    pltpu.sync_copy(item_ids.at[pl.ds(base, B_PER_W)], iidx_v)
    pltpu.sync_copy(global_bias, gb_v)

    cub = pltpu.async_copy(user_bias.at[uidx_v], ub_v, sem)
    cib = pltpu.async_copy(item_bias.at[iidx_v], ib_v, sem)

    cub.wait()
    cib.wait()
    gb = gb_v[...]

    for h in range(2):
        hoff = h * HALF
        cu = pltpu.async_copy(
            user_rows.at[uidx_v.at[pl.ds(hoff, HALF)]], urows_v, sem)
        ci = pltpu.async_copy(
            item_rows.at[iidx_v.at[pl.ds(hoff, HALF)]], irows_v, sem)
        cu.wait()
        ci.wait()

        def block_body(blk, carry):
            off = hoff + blk * LANES
            rows = blk * LANES + lax.iota(jnp.int32, LANES)
            acc = ub_v[pl.ds(off, LANES)] + ib_v[pl.ds(off, LANES)] + gb
            for d in range(EMBED_DIM):
                dcol = jnp.full((LANES,), d, jnp.int32)
                u = plsc.load_gather(urows_v, [rows, dcol])
                v = plsc.load_gather(irows_v, [rows, dcol])
                acc = acc + u * v
            out_v[pl.ds(off, LANES)] = acc
            return carry

        lax.fori_loop(0, HALF // LANES, block_body, 0, unroll=2)

    pltpu.sync_copy(out_v, out_hbm.at[pl.ds(base, B_PER_W)])


@jax.jit
def kernel(user_ids, item_ids, user_table, item_table, user_bias, item_bias,
           global_bias):
    mesh = plsc.VectorSubcoreMesh(core_axis_name="c", subcore_axis_name="s")
    run = pl.kernel(
        _mf_kernel, mesh=mesh,
        compiler_params=pltpu.CompilerParams(
            needs_layout_passes=False, use_tc_tiling_on_sc=True),
        out_type=jax.ShapeDtypeStruct((BATCH,), jnp.float32),
        scratch_types=[
            pltpu.VMEM((B_PER_W,), jnp.int32),
            pltpu.VMEM((B_PER_W,), jnp.int32),
            pltpu.VMEM((HALF, ROW_W), jnp.float32),
            pltpu.VMEM((HALF, ROW_W), jnp.float32),
            pltpu.VMEM((B_PER_W,), jnp.float32),
            pltpu.VMEM((B_PER_W,), jnp.float32),
            pltpu.VMEM((LANES,), jnp.float32),
            pltpu.VMEM((B_PER_W,), jnp.float32),
            pltpu.SemaphoreType.DMA,
        ],
    )
    pad = ((0, 0), (0, ROW_W - EMBED_DIM))
    gb16 = jnp.broadcast_to(global_bias.astype(jnp.float32), (LANES,))
    return run(user_ids.astype(jnp.int32), item_ids.astype(jnp.int32),
               jnp.pad(user_table, pad), jnp.pad(item_table, pad),
               user_bias.reshape(-1), item_bias.reshape(-1), gb16)


# speedup vs baseline: 1.5257x; 1.5257x over previous
import jax
import jax.numpy as jnp
from jax import lax
from jax.experimental import pallas as pl
from jax.experimental.pallas import tpu as pltpu
from jax.experimental.pallas import tpu_sc as plsc

NUM_CORES = 2
NUM_SUBCORES = 16
LANES = 16
NW = NUM_CORES * NUM_SUBCORES

BATCH = 16384
EMBED_DIM = 32
ROW_W = 128
B_PER_W = BATCH // NW
HALF = B_PER_W // 2


def _mf_kernel(user_ids, item_ids, user_rows, item_rows, user_bias,
               item_bias, global_bias, out_hbm,
               uidx_v, iidx_v, urows_v, irows_v, ub_v, ib_v, gb_v, out_v,
               sem):
    wid = lax.axis_index("s") * NUM_CORES + lax.axis_index("c")
    base = wid * B_PER_W

    pltpu.sync_copy(user_ids.at[pl.ds(base, B_PER_W)], uidx_v)
    pltpu.sync_copy(item_ids.at[pl.ds(base, B_PER_W)], iidx_v)
    pltpu.sync_copy(global_bias, gb_v)

    cub = pltpu.async_copy(user_bias.at[uidx_v], ub_v, sem)
    cib = pltpu.async_copy(item_bias.at[iidx_v], ib_v, sem)

    cub.wait()
    cib.wait()
    gb = gb_v[...]

    for h in range(2):
        hoff = h * HALF
        cu = pltpu.async_copy(
            user_rows.at[uidx_v.at[pl.ds(hoff, HALF)]], urows_v, sem)
        ci = pltpu.async_copy(
            item_rows.at[iidx_v.at[pl.ds(hoff, HALF)]], irows_v, sem)
        cu.wait()
        ci.wait()

        def block_body(blk, carry):
            off = hoff + blk * LANES
            rows = blk * LANES + lax.iota(jnp.int32, LANES)
            acc = ub_v[pl.ds(off, LANES)] + ib_v[pl.ds(off, LANES)] + gb
            for d in range(EMBED_DIM):
                dcol = jnp.full((LANES,), d, jnp.int32)
                u = plsc.load_gather(urows_v, [rows, dcol])
                v = plsc.load_gather(irows_v, [rows, dcol])
                acc = acc + u * v
            out_v[pl.ds(off, LANES)] = acc
            return carry

        lax.fori_loop(0, HALF // LANES, block_body, 0, unroll=2)

    pltpu.sync_copy(out_v, out_hbm.at[pl.ds(base, B_PER_W)])


_TW = 8192


def _tp_kernel(src, dst):
    dst[...] = jnp.concatenate(
        [src[...].T,
         jnp.zeros((_TW, ROW_W - EMBED_DIM), jnp.float32)], axis=1)


def _pad_transpose(table_t):
    n = table_t.shape[1]
    grid = (n + _TW - 1) // _TW
    return pl.pallas_call(
        _tp_kernel,
        grid=(grid,),
        in_specs=[pl.BlockSpec((EMBED_DIM, _TW), lambda j: (0, j))],
        out_specs=pl.BlockSpec((_TW, ROW_W), lambda j: (j, 0)),
        out_shape=jax.ShapeDtypeStruct((n, ROW_W), jnp.float32),
    )(table_t)


@jax.jit
def kernel(user_ids, item_ids, user_table, item_table, user_bias, item_bias,
           global_bias):
    mesh = plsc.VectorSubcoreMesh(core_axis_name="c", subcore_axis_name="s")
    run = pl.kernel(
        _mf_kernel, mesh=mesh,
        compiler_params=pltpu.CompilerParams(
            needs_layout_passes=False, use_tc_tiling_on_sc=True),
        out_type=jax.ShapeDtypeStruct((BATCH,), jnp.float32),
        scratch_types=[
            pltpu.VMEM((B_PER_W,), jnp.int32),
            pltpu.VMEM((B_PER_W,), jnp.int32),
            pltpu.VMEM((HALF, ROW_W), jnp.float32),
            pltpu.VMEM((HALF, ROW_W), jnp.float32),
            pltpu.VMEM((B_PER_W,), jnp.float32),
            pltpu.VMEM((B_PER_W,), jnp.float32),
            pltpu.VMEM((LANES,), jnp.float32),
            pltpu.VMEM((B_PER_W,), jnp.float32),
            pltpu.SemaphoreType.DMA,
        ],
    )
    gb16 = jnp.broadcast_to(global_bias.astype(jnp.float32), (LANES,))
    return run(user_ids.astype(jnp.int32), item_ids.astype(jnp.int32),
               _pad_transpose(user_table.T), _pad_transpose(item_table.T),
               user_bias.reshape(-1), item_bias.reshape(-1), gb16)
